# Initial kernel scaffold; baseline (speedup 1.0000x reference)
import jax
import jax.numpy as jnp
from jax.experimental import pallas as pl


def kernel(inputs, embedding):
    raise NotImplementedError("write your pallas kernel here")


# speedup vs baseline: 1.3749x; 1.3749x over previous
import functools

import jax
import jax.numpy as jnp
from jax import lax
from jax.experimental import pallas as pl
from jax.experimental.pallas import tpu as pltpu
from jax.experimental.pallas import tpu_sc as plsc

K = 8192
D = 32
NT = 8192
TB = 256
GRID = NT // TB

NC, NS, L = 2, 16, 16
NW = NC * NS
BPW = NT // NW
CH = 128
NCHUNK = BPW // CH


def _dist_body(x_ref, emb2_ref, e2_ref, idx_ref, loss_ref):
    i = pl.program_id(0)
    x = x_ref[...]
    x2 = jnp.sum(x * x, axis=1, keepdims=True)
    mm2 = jax.lax.dot_general(
        x, emb2_ref[...], (((1,), (1,)), ((), ())),
        preferred_element_type=jnp.float32)
    d = (x2 + e2_ref[...]) - mm2
    dmin = jnp.min(d, axis=1, keepdims=True)
    iota = jax.lax.broadcasted_iota(jnp.int32, (TB, K), 1)
    idx_ref[...] = jnp.min(jnp.where(d == dmin, iota, K), axis=1)

    @pl.when(i == 0)
    def _init():
        loss_ref[0, 0] = 0.0
    loss_ref[0, 0] += jnp.sum(dmin)
    @pl.when(i == GRID - 1)
    def _fini():
        loss_ref[0, 0] = 1.25 * (loss_ref[0, 0] / (NT * D))


def _encode(x, emb2, e2):
    return pl.pallas_call(
        _dist_body,
        grid=(GRID,),
        in_specs=[
            pl.BlockSpec((TB, D), lambda i: (i, 0)),
            pl.BlockSpec((K, D), lambda i: (0, 0)),
            pl.BlockSpec((1, K), lambda i: (0, 0)),
        ],
        out_specs=[
            pl.BlockSpec((TB,), lambda i: (i,)),
            pl.BlockSpec(memory_space=pltpu.SMEM),
        ],
        out_shape=[
            jax.ShapeDtypeStruct((NT,), jnp.int32),
            jax.ShapeDtypeStruct((1, 1), jnp.float32),
        ],
    )(x, emb2, e2)


def _sc_body(idx_hbm, emb_hbm, zeros_hbm, out_hbm, counts_hbm,
             idx_v, rows_v, ones_v, shared_counts, sem):
    cid = lax.axis_index("c")
    sid = lax.axis_index("s")
    wid = sid * NC + cid
    base = wid * BPW
    pltpu.sync_copy(idx_hbm.at[wid], idx_v)
    for t in range(CH // L):
        ones_v[pl.ds(L * t, L)] = jnp.ones((L,), jnp.float32)
    @pl.when(sid == 0)
    def _zero():
        pltpu.sync_copy(zeros_hbm, shared_counts)
    for j in range(NCHUNK):
        pltpu.async_copy(emb_hbm.at[idx_v.at[j]], rows_v.at[j], sem).wait()
        pltpu.sync_copy(rows_v.at[j], out_hbm.at[pl.ds(base + j * CH, CH)])
    plsc.subcore_barrier()
    for j in range(NCHUNK):
        pltpu.sync_copy(ones_v, shared_counts.at[idx_v.at[j]], add=True)
    plsc.subcore_barrier()
    @pl.when(sid == 0)
    def _writeback():
        pltpu.sync_copy(shared_counts, counts_hbm.at[cid])


_sc_gather_hist = functools.partial(
    pl.kernel,
    mesh=plsc.VectorSubcoreMesh(core_axis_name="c", subcore_axis_name="s"),
    compiler_params=pltpu.CompilerParams(use_tc_tiling_on_sc=False),
    out_type=[
        jax.ShapeDtypeStruct((NT, D), jnp.float32),
        jax.ShapeDtypeStruct((NC, K), jnp.float32),
    ],
    scratch_types=[
        pltpu.VMEM((NCHUNK, CH), jnp.int32),
        pltpu.VMEM((NCHUNK, CH, D), jnp.float32),
        pltpu.VMEM((CH,), jnp.float32),
        pltpu.VMEM_SHARED((K,), jnp.float32),
        pltpu.SemaphoreType.DMA,
    ],
)(_sc_body)


def _perp_body(counts_ref, perp_ref):
    counts = counts_ref[0, :] + counts_ref[1, :]
    p = counts * (1.0 / NT)
    perp_ref[0, 0] = jnp.exp(-jnp.sum(p * jnp.log(p + 1e-10)))


def _perplexity(counts2):
    return pl.pallas_call(
        _perp_body,
        in_specs=[pl.BlockSpec((NC, K), lambda: (0, 0))],
        out_specs=pl.BlockSpec(memory_space=pltpu.SMEM),
        out_shape=jax.ShapeDtypeStruct((1, 1), jnp.float32),
    )(counts2)


def kernel(inputs, embedding):
    n, c, h, w = inputs.shape
    x = jnp.transpose(inputs, (0, 2, 3, 1)).reshape(NT, D)
    emb2 = embedding + embedding
    e2 = jnp.sum(embedding ** 2, axis=1)[None, :]
    idx, loss = _encode(x, emb2, e2)
    q, counts2 = _sc_gather_hist(idx.reshape(NW, NCHUNK, CH), embedding,
                                 jnp.zeros((K,), jnp.float32))
    perp = _perplexity(counts2)
    quantized_st = jnp.transpose(q.reshape(n, h, w, c), (0, 3, 1, 2))
    return (loss[0, 0], quantized_st, perp[0, 0], idx.reshape(n, h, w))
